# baseline (speedup 1.0000x reference)
import functools

import jax
import jax.numpy as jnp
from jax import lax
from jax.experimental import pallas as pl
from jax.experimental.pallas import tpu as pltpu
from jax.experimental.pallas import tpu_sc as plsc

BATCH = 16
N_ITEMS = 8192
COLS_SC = 5632
SC_HALF = COLS_SC // 2
COLS_TC = N_ITEMS - COLS_SC
TC_BLK = 512
K = 8
IDX_CAP = N_ITEMS + 16


def _sc_body(x_hbm, w_hbm, out_hbm, xbuf, idxbuf, cntbuf, basebuf, acc,
             gbuf, sem0, sem1):
    c = lax.axis_index("c")
    s = lax.axis_index("s")
    iota = lax.iota(jnp.int32, 16)
    izeros = jnp.zeros((16,), jnp.int32)
    fzeros = jnp.zeros((16,), jnp.float32)
    ione = jnp.ones((16,), jnp.int32)
    col0 = c * SC_HALF

    pltpu.sync_copy(x_hbm.at[s], xbuf)

    @plsc.parallel_loop(0, IDX_CAP // 16, 1, unroll=8)
    def _(i):
        idxbuf[pl.ds(i * 16, 16)] = izeros

    @plsc.parallel_loop(0, SC_HALF // 16, 1, unroll=8)
    def _(i):
        acc[pl.ds(i * 16, 16)] = fzeros

    @plsc.parallel_loop(0, N_ITEMS // 16, 1, unroll=8)
    def _(i):
        v = xbuf[pl.ds(i * 16, 16)]
        cnt = plsc.all_reduce_population_count(v != 0.0)
        plsc.store_scatter(cntbuf, [izeros + i], cnt, mask=iota == 0)

    def pfx(j, t_v):
        cv = cntbuf[pl.ds(j * 16, 16)]
        inc = plsc.cumsum(cv)
        basebuf[pl.ds(j * 16, 16)] = t_v + inc - cv
        return t_v + (izeros + inc[15])
    t_v = lax.fori_loop(0, N_ITEMS // 256, pfx, izeros)
    total = t_v[0]

    @plsc.parallel_loop(0, N_ITEMS // 16, 1, unroll=4)
    def _(i):
        v = xbuf[pl.ds(i * 16, 16)]
        m = v != 0.0
        b16 = basebuf[pl.ds(i, 16)]
        pos = b16[0] + plsc.cumsum(jnp.where(m, ione, izeros)) - 1
        plsc.store_scatter(idxbuf, [pos], i * 16 + iota, mask=m)
    nchunks = (total + (K - 1)) // K

    def start(g, nb, sem):
        pltpu.async_copy(
            w_hbm.at[idxbuf.at[pl.ds(g * K, K)], pl.ds(col0, SC_HALF)],
            gbuf.at[nb], sem)

    def wait(nb, sem):
        pltpu.make_async_copy(
            w_hbm.at[pl.ds(0, K), pl.ds(col0, SC_HALF)],
            gbuf.at[nb], sem).wait()

    def process(g, nb):
        for r in range(K):
            @pl.when(g * K + r < total)
            def _():
                @plsc.parallel_loop(0, SC_HALF // 16, 1, unroll=8)
                def _(i):
                    o = i * 16
                    plsc.addupdate(acc.at[pl.ds(o, 16)],
                                   gbuf[nb, r, pl.ds(o, 16)])

    @pl.when(nchunks > 0)
    def _():
        start(0, 0, sem0)

    def ring(g, _):
        @pl.when(lax.rem(g, 2) == 0)
        def _():
            @pl.when(g + 1 < nchunks)
            def _():
                start(g + 1, 1, sem1)
            wait(0, sem0)
            process(g, 0)

        @pl.when(lax.rem(g, 2) == 1)
        def _():
            @pl.when(g + 1 < nchunks)
            def _():
                start(g + 1, 0, sem0)
            wait(1, sem1)
            process(g, 1)
        return 0
    lax.fori_loop(0, nchunks, ring, 0)

    pltpu.sync_copy(acc, out_hbm.at[s, pl.ds(col0, SC_HALF)])


_taylor_sc = functools.partial(
    pl.kernel,
    out_type=jax.ShapeDtypeStruct((BATCH, COLS_SC), jnp.float32),
    mesh=plsc.VectorSubcoreMesh(core_axis_name="c", subcore_axis_name="s"),
    compiler_params=pltpu.CompilerParams(needs_layout_passes=False),
    scratch_types=[
        pltpu.VMEM((N_ITEMS,), jnp.float32),
        pltpu.VMEM((IDX_CAP,), jnp.int32),
        pltpu.VMEM((N_ITEMS // 16 + 16,), jnp.int32),
        pltpu.VMEM((N_ITEMS // 16 + 16,), jnp.int32),
        pltpu.VMEM((SC_HALF,), jnp.float32),
        pltpu.VMEM((2, K, SC_HALF), jnp.float32),
        pltpu.SemaphoreType.DMA,
        pltpu.SemaphoreType.DMA,
    ],
)(_sc_body)


def _tc_body(x_ref, w_ref, o_ref):
    o_ref[...] = jnp.dot(x_ref[...], w_ref[...],
                         preferred_element_type=jnp.float32)


def _tc_mm(X_batch, W):
    return pl.pallas_call(
        _tc_body,
        grid=(COLS_TC // TC_BLK,),
        in_specs=[
            pl.BlockSpec((BATCH, N_ITEMS), lambda j: (0, 0)),
            pl.BlockSpec((N_ITEMS, TC_BLK),
                         lambda j: (0, COLS_SC // TC_BLK + j)),
        ],
        out_specs=pl.BlockSpec((BATCH, TC_BLK), lambda j: (0, j)),
        out_shape=jax.ShapeDtypeStruct((BATCH, COLS_TC), jnp.float32),
    )(X_batch, W)


@jax.jit
def kernel(X_batch, W):
    tc_out = _tc_mm(X_batch, W)
    sc_out = _taylor_sc(X_batch, W)
    return jnp.concatenate([sc_out, tc_out], axis=1)


# speedup vs baseline: 1.0478x; 1.0478x over previous
import functools

import jax
import jax.numpy as jnp
from jax import lax
from jax.experimental import pallas as pl
from jax.experimental.pallas import tpu as pltpu
from jax.experimental.pallas import tpu_sc as plsc

BATCH = 16
N_ITEMS = 8192
COLS_SC = 5120
SC_HALF = COLS_SC // 2
COLS_TC = N_ITEMS - COLS_SC
TC_BLK = 512
K = 8
IDX_CAP = N_ITEMS + 16


def _sc_body(x_hbm, w_hbm, out_hbm, xbuf, idxbuf, cntbuf, basebuf, acc,
             gbuf, sem0, sem1):
    c = lax.axis_index("c")
    s = lax.axis_index("s")
    iota = lax.iota(jnp.int32, 16)
    izeros = jnp.zeros((16,), jnp.int32)
    fzeros = jnp.zeros((16,), jnp.float32)
    ione = jnp.ones((16,), jnp.int32)
    col0 = c * SC_HALF

    pltpu.sync_copy(x_hbm.at[s], xbuf)

    @plsc.parallel_loop(0, IDX_CAP // 16, 1, unroll=8)
    def _(i):
        idxbuf[pl.ds(i * 16, 16)] = izeros

    @plsc.parallel_loop(0, SC_HALF // 16, 1, unroll=8)
    def _(i):
        acc[pl.ds(i * 16, 16)] = fzeros

    @plsc.parallel_loop(0, N_ITEMS // 16, 1, unroll=8)
    def _(i):
        v = xbuf[pl.ds(i * 16, 16)]
        cnt = plsc.all_reduce_population_count(v != 0.0)
        plsc.store_scatter(cntbuf, [izeros + i], cnt, mask=iota == 0)

    def pfx(j, t_v):
        cv = cntbuf[pl.ds(j * 16, 16)]
        inc = plsc.cumsum(cv)
        basebuf[pl.ds(j * 16, 16)] = t_v + inc - cv
        return t_v + (izeros + inc[15])
    t_v = lax.fori_loop(0, N_ITEMS // 256, pfx, izeros)
    total = t_v[0]

    @plsc.parallel_loop(0, N_ITEMS // 16, 1, unroll=4)
    def _(i):
        v = xbuf[pl.ds(i * 16, 16)]
        m = v != 0.0
        b16 = basebuf[pl.ds(i, 16)]
        pos = b16[0] + plsc.cumsum(jnp.where(m, ione, izeros)) - 1
        plsc.store_scatter(idxbuf, [pos], i * 16 + iota, mask=m)
    nchunks = (total + (K - 1)) // K

    def start(g, nb, sem):
        pltpu.async_copy(
            w_hbm.at[idxbuf.at[pl.ds(g * K, K)], pl.ds(col0, SC_HALF)],
            gbuf.at[nb], sem)

    def wait(nb, sem):
        pltpu.make_async_copy(
            w_hbm.at[pl.ds(0, K), pl.ds(col0, SC_HALF)],
            gbuf.at[nb], sem).wait()

    def process(g, nb):
        for r in range(K):
            @pl.when(g * K + r < total)
            def _():
                @plsc.parallel_loop(0, SC_HALF // 16, 1, unroll=8)
                def _(i):
                    o = i * 16
                    plsc.addupdate(acc.at[pl.ds(o, 16)],
                                   gbuf[nb, r, pl.ds(o, 16)])

    @pl.when(nchunks > 0)
    def _():
        start(0, 0, sem0)

    def ring(g, _):
        @pl.when(lax.rem(g, 2) == 0)
        def _():
            @pl.when(g + 1 < nchunks)
            def _():
                start(g + 1, 1, sem1)
            wait(0, sem0)
            process(g, 0)

        @pl.when(lax.rem(g, 2) == 1)
        def _():
            @pl.when(g + 1 < nchunks)
            def _():
                start(g + 1, 0, sem0)
            wait(1, sem1)
            process(g, 1)
        return 0
    lax.fori_loop(0, nchunks, ring, 0)

    pltpu.sync_copy(acc, out_hbm.at[s, pl.ds(col0, SC_HALF)])


_taylor_sc = functools.partial(
    pl.kernel,
    out_type=jax.ShapeDtypeStruct((BATCH, COLS_SC), jnp.float32),
    mesh=plsc.VectorSubcoreMesh(core_axis_name="c", subcore_axis_name="s"),
    compiler_params=pltpu.CompilerParams(needs_layout_passes=False),
    scratch_types=[
        pltpu.VMEM((N_ITEMS,), jnp.float32),
        pltpu.VMEM((IDX_CAP,), jnp.int32),
        pltpu.VMEM((N_ITEMS // 16 + 16,), jnp.int32),
        pltpu.VMEM((N_ITEMS // 16 + 16,), jnp.int32),
        pltpu.VMEM((SC_HALF,), jnp.float32),
        pltpu.VMEM((2, K, SC_HALF), jnp.float32),
        pltpu.SemaphoreType.DMA,
        pltpu.SemaphoreType.DMA,
    ],
)(_sc_body)


def _tc_body(x_ref, w_ref, o_ref):
    o_ref[...] = jnp.dot(x_ref[...], w_ref[...],
                         preferred_element_type=jnp.float32)


def _tc_mm(X_batch, W):
    return pl.pallas_call(
        _tc_body,
        grid=(COLS_TC // TC_BLK,),
        in_specs=[
            pl.BlockSpec((BATCH, N_ITEMS), lambda j: (0, 0)),
            pl.BlockSpec((N_ITEMS, TC_BLK),
                         lambda j: (0, COLS_SC // TC_BLK + j)),
        ],
        out_specs=pl.BlockSpec((BATCH, TC_BLK), lambda j: (0, j)),
        out_shape=jax.ShapeDtypeStruct((BATCH, COLS_TC), jnp.float32),
    )(X_batch, W)


@jax.jit
def kernel(X_batch, W):
    tc_out = _tc_mm(X_batch, W)
    sc_out = _taylor_sc(X_batch, W)
    return jnp.concatenate([sc_out, tc_out], axis=1)
